# baseline (speedup 1.0000x reference)
import functools

import jax
import jax.numpy as jnp
from jax import lax
from jax.experimental import pallas as pl
from jax.experimental.pallas import tpu as pltpu
from jax.experimental.pallas import tpu_sc as plsc

B = 1024
L1 = 26
L2 = 20
D = 32
NBLK = L1 * L2
NC = 2
NS = 16
NW = NC * NS
ROUNDS = NBLK // NW
TAIL = NBLK - ROUNDS * NW

_mesh = plsc.VectorSubcoreMesh(core_axis_name="c", subcore_axis_name="s")


@functools.partial(
    pl.kernel,
    mesh=_mesh,
    out_type=jax.ShapeDtypeStruct((NBLK, 4, 8, 8, 128), jnp.float32),
    scratch_types=[
        pltpu.VMEM((2, B), jnp.int32),
        pltpu.VMEM((B, D), jnp.float32),
        pltpu.VMEM((B, D), jnp.float32),
        pltpu.VMEM((4, 8, 8, 128), jnp.float32),
        pltpu.SemaphoreType.DMA,
    ],
    compiler_params=pltpu.CompilerParams(
        use_tc_tiling_on_sc=False, needs_layout_passes=False
    ),
)
def _gather_kernel(table_hbm, idx_hbm, out_hbm, idx_v, rows_v0, rows_v1, t_v, gsem):
    w = lax.axis_index("s") * NC + lax.axis_index("c")
    iota16 = lax.iota(jnp.int32, 16)
    rows_bufs = (rows_v0, rows_v1)

    def idx_load(slot, k):
        pltpu.sync_copy(idx_hbm.at[pl.ds(k * B, B)], idx_v.at[slot])

    def fire(slot):
        return pltpu.async_copy(table_hbm.at[idx_v.at[slot]], rows_bufs[slot], gsem)

    def transpose(slot):
        rows = rows_bufs[slot]

        def body(m, carry):
            rowvec = (m // 8) * 128 + (m % 8) * 16 + iota16
            for dt in range(4):
                for dr in range(8):
                    d = dt * 8 + dr
                    vals = plsc.load_gather(
                        rows, [rowvec, jnp.full((16,), d, jnp.int32)]
                    )
                    t_v[dt, m // 8, dr, pl.ds((m % 8) * 16, 16)] = vals
            return carry

        lax.fori_loop(0, 64, body, 0)

    idx_load(0, w)
    gathers = [fire(0), None]
    idx_load(1, w + NW)
    gathers[1] = fire(1)

    for r in range(ROUNDS):
        slot = r % 2
        gathers[slot].wait()
        transpose(slot)
        nxt = r + 2
        if nxt < ROUNDS:
            idx_load(slot, w + NW * nxt)
            gathers[slot] = fire(slot)
        pltpu.sync_copy(t_v, out_hbm.at[w + NW * r])

    @pl.when(w < TAIL)
    def _():
        k = ROUNDS * NW + w
        idx_load(0, k)
        pltpu.async_copy(table_hbm.at[idx_v.at[0]], rows_v0, gsem).wait()
        transpose(0)
        pltpu.sync_copy(t_v, out_hbm.at[k])


def kernel(x, W):
    xt = jnp.transpose(x, (1, 2, 0)).reshape(-1).astype(jnp.int32)
    pal = _gather_kernel(W, xt)
    pal6 = pal.reshape(L1, L2, 4, 8, 8, 128)
    out = jnp.transpose(pal6, (3, 5, 0, 1, 2, 4)).reshape(B, L1, L2, D)
    return out


# speedup vs baseline: 1.4204x; 1.4204x over previous
import functools

import jax
import jax.numpy as jnp
from jax import lax
from jax.experimental import pallas as pl
from jax.experimental.pallas import tpu as pltpu
from jax.experimental.pallas import tpu_sc as plsc

B = 1024
L1 = 26
L2 = 20
D = 32
NBLK = L1 * L2
NC = 2
NS = 16
NW = NC * NS
ROUNDS = NBLK // NW
TAIL = NBLK - ROUNDS * NW

_mesh = plsc.VectorSubcoreMesh(core_axis_name="c", subcore_axis_name="s")


@functools.partial(
    pl.kernel,
    mesh=_mesh,
    out_type=jax.ShapeDtypeStruct((NBLK, 4, 8, 8, 128), jnp.float32),
    scratch_types=[
        pltpu.VMEM((2, B), jnp.int32),
        pltpu.VMEM((B, D), jnp.float32),
        pltpu.VMEM((B, D), jnp.float32),
        pltpu.VMEM((4, 8, 8, 128), jnp.float32),
        pltpu.SemaphoreType.DMA,
    ],
    compiler_params=pltpu.CompilerParams(
        use_tc_tiling_on_sc=False, needs_layout_passes=False
    ),
)
def _gather_kernel(table_hbm, idx_hbm, out_hbm, idx_v, rows_v0, rows_v1, t_v, gsem):
    w = lax.axis_index("s") * NC + lax.axis_index("c")
    iota16 = lax.iota(jnp.int32, 16)
    rows_bufs = (rows_v0, rows_v1)

    def idx_load(slot, k):
        pltpu.sync_copy(idx_hbm.at[pl.ds(k * B, B)], idx_v.at[slot])

    def fire(slot):
        return pltpu.async_copy(table_hbm.at[idx_v.at[slot]], rows_bufs[slot], gsem)

    def transpose(slot):
        rows = rows_bufs[slot]

        def body(m, carry):
            rowvec = m * 16 + iota16
            blvec = (m % 8) * 16 + iota16
            bhvec = jnp.full((16,), 0, jnp.int32) + m // 8
            for d0 in range(D):
                dvec = (iota16 + d0) % D
                vals = plsc.load_gather(rows, [rowvec, dvec])
                plsc.store_scatter(t_v, [dvec // 8, bhvec, dvec % 8, blvec], vals)
            return carry

        lax.fori_loop(0, 64, body, 0)

    idx_load(0, w)
    gathers = [fire(0), None]
    idx_load(1, w + NW)
    gathers[1] = fire(1)

    for r in range(ROUNDS):
        slot = r % 2
        gathers[slot].wait()
        transpose(slot)
        nxt = r + 2
        if nxt < ROUNDS:
            idx_load(slot, w + NW * nxt)
            gathers[slot] = fire(slot)
        pltpu.sync_copy(t_v, out_hbm.at[w + NW * r])

    @pl.when(w < TAIL)
    def _():
        k = ROUNDS * NW + w
        idx_load(0, k)
        pltpu.async_copy(table_hbm.at[idx_v.at[0]], rows_v0, gsem).wait()
        transpose(0)
        pltpu.sync_copy(t_v, out_hbm.at[k])


def kernel(x, W):
    xt = jnp.transpose(x, (1, 2, 0)).reshape(-1).astype(jnp.int32)
    pal = _gather_kernel(W, xt)
    pal6 = pal.reshape(L1, L2, 4, 8, 8, 128)
    out = jnp.transpose(pal6, (3, 5, 0, 1, 2, 4)).reshape(B, L1, L2, D)
    return out
